# baseline (speedup 1.0000x reference)
import functools

import jax
import jax.numpy as jnp
from jax.experimental import pallas as pl

INTERPRET = False

N_EDGE_BLK = 3200
N_NODE_BLK = 2000


def _ln(x, scale, bias):
    mu = jnp.mean(x, axis=-1, keepdims=True)
    var = jnp.mean((x - mu) ** 2, axis=-1, keepdims=True)
    return (x - mu) * jax.lax.rsqrt(var + 1e-5) * scale + bias


def _enc_body(x_ref, w1, b1, w2, b2, w3, b3, lns, lnb, o_ref):
    h = jnp.maximum(jnp.dot(x_ref[...], w1[...],
                            preferred_element_type=jnp.float32, precision=jax.lax.Precision.HIGHEST) + b1[...], 0.0)
    h = jnp.maximum(jnp.dot(h, w2[...],
                            preferred_element_type=jnp.float32, precision=jax.lax.Precision.HIGHEST) + b2[...], 0.0)
    h = jnp.dot(h, w3[...], preferred_element_type=jnp.float32, precision=jax.lax.Precision.HIGHEST) + b3[...]
    o_ref[...] = _ln(h, lns[...], lnb[...])


def _dec_body(x_ref, w1, b1, w2, b2, w3, b3, o_ref):
    h = jnp.maximum(jnp.dot(x_ref[...], w1[...],
                            preferred_element_type=jnp.float32, precision=jax.lax.Precision.HIGHEST) + b1[...], 0.0)
    h = jnp.maximum(jnp.dot(h, w2[...],
                            preferred_element_type=jnp.float32, precision=jax.lax.Precision.HIGHEST) + b2[...], 0.0)
    o_ref[...] = jnp.dot(h, w3[...], preferred_element_type=jnp.float32, precision=jax.lax.Precision.HIGHEST) + b3[...]


def _edge_proc_body(e_ref, g_ref, w1, b1, w2, b2, w3, b3, lns, lnb, o_ref):
    h = jnp.maximum(jnp.dot(e_ref[...], w1[...],
                            preferred_element_type=jnp.float32, precision=jax.lax.Precision.HIGHEST)
                    + g_ref[...] + b1[...], 0.0)
    h = jnp.maximum(jnp.dot(h, w2[...],
                            preferred_element_type=jnp.float32, precision=jax.lax.Precision.HIGHEST) + b2[...], 0.0)
    h = jnp.dot(h, w3[...], preferred_element_type=jnp.float32, precision=jax.lax.Precision.HIGHEST) + b3[...]
    o_ref[...] = _ln(h, lns[...], lnb[...]) + e_ref[...]


def _node_proc_body(x_ref, a_ref, w1x, w1a, b1, w2, b2, w3, b3, lns, lnb,
                    o_ref):
    h = jnp.maximum(jnp.dot(x_ref[...], w1x[...],
                            preferred_element_type=jnp.float32, precision=jax.lax.Precision.HIGHEST)
                    + jnp.dot(a_ref[...], w1a[...],
                              preferred_element_type=jnp.float32, precision=jax.lax.Precision.HIGHEST)
                    + b1[...], 0.0)
    h = jnp.maximum(jnp.dot(h, w2[...],
                            preferred_element_type=jnp.float32, precision=jax.lax.Precision.HIGHEST) + b2[...], 0.0)
    h = jnp.dot(h, w3[...], preferred_element_type=jnp.float32, precision=jax.lax.Precision.HIGHEST) + b3[...]
    o_ref[...] = _ln(h, lns[...], lnb[...]) + x_ref[...]


def _proj_body(x_ref, ws, o_ref):
    o_ref[...] = jnp.dot(x_ref[...], ws[...],
                         preferred_element_type=jnp.float32, precision=jax.lax.Precision.HIGHEST)


def _row_call(body, n_rows, blk, row_args, full_args, out_dims):
    grid = (n_rows // blk,)
    in_specs = []
    for a in row_args:
        in_specs.append(pl.BlockSpec((blk, a.shape[1]), lambda i: (i, 0)))
    for a in full_args:
        in_specs.append(
            pl.BlockSpec(a.shape, lambda i: tuple(0 for _ in a.shape)))
    if isinstance(out_dims, tuple):
        out_shape = jax.ShapeDtypeStruct((n_rows, out_dims[1]), jnp.float32)
        out_spec = pl.BlockSpec((blk, out_dims[1]), lambda i: (i, 0))
    return pl.pallas_call(
        body, grid=grid, in_specs=in_specs, out_specs=out_spec,
        out_shape=out_shape, interpret=INTERPRET,
    )(*row_args, *full_args)


def _mlp_params(p):
    Ws, bs = p["Ws"], p["bs"]
    out = []
    for W, b in zip(Ws, bs):
        out.append(W)
        out.append(b.reshape(1, -1))
    if "ln_scale" in p:
        out.append(p["ln_scale"].reshape(1, -1))
        out.append(p["ln_bias"].reshape(1, -1))
    return out


def kernel(node_features, edge_features, edge_index, params):
    n_nodes = node_features.shape[0]
    n_edges = edge_features.shape[0]
    src = edge_index[0]
    dst = edge_index[1]

    e = _row_call(_enc_body, n_edges, N_EDGE_BLK, [edge_features],
                  _mlp_params(params["edge_enc"]), (None, 128))
    x = _row_call(_enc_body, n_nodes, N_NODE_BLK, [node_features],
                  _mlp_params(params["node_enc"]), (None, 128))

    for layer in params["processor"]:
        ep = layer["edge_mlp"]
        np_ = layer["node_mlp"]
        W1 = ep["Ws"][0]
        W1e = W1[:128]
        Wsd = jnp.concatenate([W1[128:256], W1[256:384]], axis=1)
        xp = _row_call(_proj_body, n_nodes, N_NODE_BLK, [x], [Wsd],
                       (None, 256))
        xs = xp[:, :128]
        xd = xp[:, 128:]
        g = jnp.take(xs, src, axis=0) + jnp.take(xd, dst, axis=0)
        ep_rest = [W1e, ep["bs"][0].reshape(1, -1),
                   ep["Ws"][1], ep["bs"][1].reshape(1, -1),
                   ep["Ws"][2], ep["bs"][2].reshape(1, -1),
                   ep["ln_scale"].reshape(1, -1), ep["ln_bias"].reshape(1, -1)]
        e = _row_call(_edge_proc_body, n_edges, N_EDGE_BLK, [e, g], ep_rest,
                      (None, 128))
        agg = jax.ops.segment_sum(e, dst, num_segments=n_nodes)
        nW1 = np_["Ws"][0]
        np_rest = [nW1[:128], nW1[128:], np_["bs"][0].reshape(1, -1),
                   np_["Ws"][1], np_["bs"][1].reshape(1, -1),
                   np_["Ws"][2], np_["bs"][2].reshape(1, -1),
                   np_["ln_scale"].reshape(1, -1),
                   np_["ln_bias"].reshape(1, -1)]
        x = _row_call(_node_proc_body, n_nodes, N_NODE_BLK, [x, agg], np_rest,
                      (None, 128))

    out = _row_call(_dec_body, n_nodes, N_NODE_BLK, [x],
                    _mlp_params(params["node_dec"]), (None, 3))
    return out


# speedup vs baseline: 1.7905x; 1.7905x over previous
import functools

import jax
import jax.numpy as jnp
from jax import lax
from jax.experimental import pallas as pl
from jax.experimental.pallas import tpu as pltpu
from jax.experimental.pallas import tpu_sc as plsc

INTERPRET = False

NW = 32
K = 128
NCHUNK = 80
C = NCHUNK * K
NE_PAD = NW * C

N_EDGE_BLK = 2560
N_NODE_BLK = 2000

HI = jax.lax.Precision.HIGHEST


def _ln(x, scale, bias):
    mu = jnp.mean(x, axis=-1, keepdims=True)
    var = jnp.mean((x - mu) ** 2, axis=-1, keepdims=True)
    return (x - mu) * jax.lax.rsqrt(var + 1e-5) * scale + bias


def _enc_body(x_ref, w1, b1, w2, b2, w3, b3, lns, lnb, o_ref):
    h = jnp.maximum(jnp.dot(x_ref[...], w1[...],
                            preferred_element_type=jnp.float32,
                            precision=HI) + b1[...], 0.0)
    h = jnp.maximum(jnp.dot(h, w2[...], preferred_element_type=jnp.float32,
                            precision=HI) + b2[...], 0.0)
    h = jnp.dot(h, w3[...], preferred_element_type=jnp.float32,
                precision=HI) + b3[...]
    o_ref[...] = _ln(h, lns[...], lnb[...])


def _dec_body(x_ref, w1, b1, w2, b2, w3, b3, o_ref):
    h = jnp.maximum(jnp.dot(x_ref[...], w1[...],
                            preferred_element_type=jnp.float32,
                            precision=HI) + b1[...], 0.0)
    h = jnp.maximum(jnp.dot(h, w2[...], preferred_element_type=jnp.float32,
                            precision=HI) + b2[...], 0.0)
    o_ref[...] = jnp.dot(h, w3[...], preferred_element_type=jnp.float32,
                         precision=HI) + b3[...]


def _edge_proc_body(e_ref, gs_ref, gd_ref, w1, b1, w2, b2, w3, b3, lns, lnb,
                    o_ref):
    h = jnp.maximum(jnp.dot(e_ref[...], w1[...],
                            preferred_element_type=jnp.float32, precision=HI)
                    + gs_ref[...] + gd_ref[...] + b1[...], 0.0)
    h = jnp.maximum(jnp.dot(h, w2[...], preferred_element_type=jnp.float32,
                            precision=HI) + b2[...], 0.0)
    h = jnp.dot(h, w3[...], preferred_element_type=jnp.float32,
                precision=HI) + b3[...]
    o_ref[...] = _ln(h, lns[...], lnb[...]) + e_ref[...]


def _node_proc_body(x_ref, p0_ref, p1_ref, w1x, w1a, b1, w2, b2, w3, b3,
                    lns, lnb, o_ref):
    agg = p0_ref[...] + p1_ref[...]
    h = jnp.maximum(jnp.dot(x_ref[...], w1x[...],
                            preferred_element_type=jnp.float32, precision=HI)
                    + jnp.dot(agg, w1a[...],
                              preferred_element_type=jnp.float32,
                              precision=HI)
                    + b1[...], 0.0)
    h = jnp.maximum(jnp.dot(h, w2[...], preferred_element_type=jnp.float32,
                            precision=HI) + b2[...], 0.0)
    h = jnp.dot(h, w3[...], preferred_element_type=jnp.float32,
                precision=HI) + b3[...]
    o_ref[...] = _ln(h, lns[...], lnb[...]) + x_ref[...]


def _proj_body(x_ref, ws, os_ref, od_ref):
    h = jnp.dot(x_ref[...], ws[...], preferred_element_type=jnp.float32,
                precision=HI)
    os_ref[...] = h[:, :128]
    od_ref[...] = h[:, 128:]


def _row_call(body, n_rows, blk, row_args, full_args, out_dim):
    grid = (n_rows // blk,)
    in_specs = []
    for a in row_args:
        in_specs.append(pl.BlockSpec((blk, a.shape[1]), lambda i: (i, 0)))
    for a in full_args:
        in_specs.append(
            pl.BlockSpec(a.shape, lambda i: tuple(0 for _ in a.shape)))
    out_shape = jax.ShapeDtypeStruct((n_rows, out_dim), jnp.float32)
    out_spec = pl.BlockSpec((blk, out_dim), lambda i: (i, 0))
    return pl.pallas_call(
        body, grid=grid, in_specs=in_specs, out_specs=out_spec,
        out_shape=out_shape, interpret=INTERPRET,
    )(*row_args, *full_args)


def _proj_call(x, ws):
    n_rows = x.shape[0]
    grid = (n_rows // N_NODE_BLK,)
    out_sds = jax.ShapeDtypeStruct((n_rows, 128), jnp.float32)
    out_spec = pl.BlockSpec((N_NODE_BLK, 128), lambda i: (i, 0))
    return pl.pallas_call(
        _proj_body, grid=grid,
        in_specs=[pl.BlockSpec((N_NODE_BLK, 128), lambda i: (i, 0)),
                  pl.BlockSpec(ws.shape, lambda i: (0, 0))],
        out_specs=[out_spec, out_spec],
        out_shape=[out_sds, out_sds], interpret=INTERPRET,
    )(x, ws)


def _make_sc_gather():
    mesh = plsc.VectorSubcoreMesh(core_axis_name="c", subcore_axis_name="s")
    gshape = jax.ShapeDtypeStruct((NE_PAD, 128), jnp.float32)

    @functools.partial(
        pl.kernel,
        out_type=[gshape, gshape],
        mesh=mesh,
        scratch_types=[
            pltpu.VMEM((NCHUNK, K), jnp.int32),
            pltpu.VMEM((NCHUNK, K), jnp.int32),
            pltpu.VMEM((2, K, 128), jnp.float32),
            pltpu.VMEM((2, K, 128), jnp.float32),
            pltpu.SemaphoreType.DMA,
            pltpu.SemaphoreType.DMA,
            pltpu.SemaphoreType.DMA,
            pltpu.SemaphoreType.DMA,
        ])
    def gather_k(xs_hbm, xd_hbm, src_hbm, dst_hbm, gs_hbm, gd_hbm,
                 src_v, dst_v, bufa, bufb, sema, semb, semoa, semob):
        cid = lax.axis_index("c")
        sid = lax.axis_index("s")
        wid = sid * 2 + cid
        pltpu.sync_copy(src_hbm.at[wid], src_v)
        pltpu.sync_copy(dst_hbm.at[wid], dst_v)
        base = wid * C

        def chunk(j, _):
            cpa = pltpu.async_copy(xs_hbm.at[src_v.at[j]], bufa.at[0], sema)
            cpb = pltpu.async_copy(xd_hbm.at[dst_v.at[j]], bufb.at[0], semb)
            cpa.wait()
            cpb.wait()
            rows = pl.ds(base + j * K, K)
            oa = pltpu.async_copy(bufa.at[0], gs_hbm.at[rows], semoa)
            ob = pltpu.async_copy(bufb.at[0], gd_hbm.at[rows], semob)
            oa.wait()
            ob.wait()
            return 0

        lax.fori_loop(0, NCHUNK, chunk, 0)

    return gather_k


def _make_sc_scatter(n_nodes, n_acc):
    mesh = plsc.VectorSubcoreMesh(core_axis_name="c", subcore_axis_name="s")
    rows_per_tile = n_acc // 16
    zr = rows_per_tile // 4

    @functools.partial(
        pl.kernel,
        out_type=jax.ShapeDtypeStruct((2, n_acc, 128), jnp.float32),
        mesh=mesh,
        scratch_types=[
            pltpu.VMEM((NCHUNK, K), jnp.int32),
            pltpu.VMEM((K, 128), jnp.float32),
            pltpu.VMEM((zr, 128), jnp.float32),
            pltpu.VMEM_SHARED((n_acc, 128), jnp.float32),
            pltpu.SemaphoreType.DMA,
        ])
    def scatter_k(e_hbm, dst_hbm, out_hbm, dst_v, ebuf, zbuf, acc, sem):
        cid = lax.axis_index("c")
        sid = lax.axis_index("s")
        wid = sid * 2 + cid
        pltpu.sync_copy(dst_hbm.at[wid], dst_v)

        def zrow(r, _):
            for c2 in range(8):
                zbuf[r, pl.ds(c2 * 16, 16)] = jnp.zeros((16,), jnp.float32)
            return 0

        lax.fori_loop(0, zr, zrow, 0)
        for t in range(4):
            pltpu.sync_copy(
                zbuf, acc.at[pl.ds(sid * rows_per_tile + t * zr, zr)])
        plsc.subcore_barrier()

        def chunk(j, _):
            cp = pltpu.async_copy(
                e_hbm.at[pl.ds(wid * C + j * K, K)], ebuf, sem)
            cp.wait()
            pltpu.sync_copy(ebuf, acc.at[dst_v.at[j]], add=True)
            return 0

        lax.fori_loop(0, NCHUNK, chunk, 0)
        plsc.subcore_barrier()
        rows = pl.ds(sid * rows_per_tile, rows_per_tile)
        pltpu.sync_copy(acc.at[rows], out_hbm.at[cid, rows])

    return scatter_k


def _mlp_params(p):
    out = []
    for W, b in zip(p["Ws"], p["bs"]):
        out.append(W)
        out.append(b.reshape(1, -1))
    if "ln_scale" in p:
        out.append(p["ln_scale"].reshape(1, -1))
        out.append(p["ln_bias"].reshape(1, -1))
    return out


def kernel(node_features, edge_features, edge_index, params):
    n_nodes = node_features.shape[0]
    n_edges = edge_features.shape[0]
    n_acc = (n_nodes + 128) // 128 * 128
    src = edge_index[0]
    dst = edge_index[1]

    pad = NE_PAD - n_edges
    zpad = jnp.zeros((pad,), jnp.int32)
    src_g = jnp.concatenate([src, zpad]).reshape(NW, NCHUNK, K)
    dst_g = jnp.concatenate([dst, zpad]).reshape(NW, NCHUNK, K)
    dst_s = jnp.concatenate(
        [dst, jnp.full((pad,), n_nodes, jnp.int32)]).reshape(NW, NCHUNK, K)
    ef_pad = jnp.concatenate(
        [edge_features, jnp.zeros((pad, edge_features.shape[1]),
                                  jnp.float32)], axis=0)

    sc_gather = _make_sc_gather()
    sc_scatter = _make_sc_scatter(n_nodes, n_acc)

    e = _row_call(_enc_body, NE_PAD, N_EDGE_BLK, [ef_pad],
                  _mlp_params(params["edge_enc"]), 128)
    x = _row_call(_enc_body, n_nodes, N_NODE_BLK, [node_features],
                  _mlp_params(params["node_enc"]), 128)

    for layer in params["processor"]:
        ep = layer["edge_mlp"]
        npm = layer["node_mlp"]
        W1 = ep["Ws"][0]
        W1e = W1[:128]
        Wsd = jnp.concatenate([W1[128:256], W1[256:384]], axis=1)
        xs, xd = _proj_call(x, Wsd)
        gs, gd = sc_gather(xs, xd, src_g, dst_g)
        ep_rest = [W1e, ep["bs"][0].reshape(1, -1),
                   ep["Ws"][1], ep["bs"][1].reshape(1, -1),
                   ep["Ws"][2], ep["bs"][2].reshape(1, -1),
                   ep["ln_scale"].reshape(1, -1),
                   ep["ln_bias"].reshape(1, -1)]
        e = _row_call(_edge_proc_body, NE_PAD, N_EDGE_BLK, [e, gs, gd],
                      ep_rest, 128)
        part = sc_scatter(e, dst_s)
        nW1 = npm["Ws"][0]
        np_rest = [nW1[:128], nW1[128:], npm["bs"][0].reshape(1, -1),
                   npm["Ws"][1], npm["bs"][1].reshape(1, -1),
                   npm["Ws"][2], npm["bs"][2].reshape(1, -1),
                   npm["ln_scale"].reshape(1, -1),
                   npm["ln_bias"].reshape(1, -1)]
        x = _row_call(_node_proc_body, n_nodes, N_NODE_BLK,
                      [x, part[0], part[1]], np_rest, 128)

    out = _row_call(_dec_body, n_nodes, N_NODE_BLK, [x],
                    _mlp_params(params["node_dec"]), 3)
    return out


# speedup vs baseline: 2.1123x; 1.1797x over previous
import functools

import jax
import jax.numpy as jnp
from jax import lax
from jax.experimental import pallas as pl
from jax.experimental.pallas import tpu as pltpu
from jax.experimental.pallas import tpu_sc as plsc

INTERPRET = False

NW = 32
K = 128
NCHUNK = 80
C = NCHUNK * K
NE_PAD = NW * C

N_EDGE_BLK = 2560
N_NODE_BLK = 2000

HI = jax.lax.Precision.HIGHEST


def _ln(x, scale, bias):
    mu = jnp.mean(x, axis=-1, keepdims=True)
    var = jnp.mean((x - mu) ** 2, axis=-1, keepdims=True)
    return (x - mu) * jax.lax.rsqrt(var + 1e-5) * scale + bias


def _enc_body(x_ref, w1, b1, w2, b2, w3, b3, lns, lnb, o_ref):
    h = jnp.maximum(jnp.dot(x_ref[...], w1[...],
                            preferred_element_type=jnp.float32,
                           ) + b1[...], 0.0)
    h = jnp.maximum(jnp.dot(h, w2[...], preferred_element_type=jnp.float32,
                           ) + b2[...], 0.0)
    h = jnp.dot(h, w3[...], preferred_element_type=jnp.float32,
               ) + b3[...]
    o_ref[...] = _ln(h, lns[...], lnb[...])


def _dec_body(x_ref, w1, b1, w2, b2, w3, b3, o_ref):
    h = jnp.maximum(jnp.dot(x_ref[...], w1[...],
                            preferred_element_type=jnp.float32,
                           ) + b1[...], 0.0)
    h = jnp.maximum(jnp.dot(h, w2[...], preferred_element_type=jnp.float32,
                           ) + b2[...], 0.0)
    o_ref[...] = jnp.dot(h, w3[...], preferred_element_type=jnp.float32,
                        ) + b3[...]


def _edge_proc_body(e_ref, gs_ref, gd_ref, w1, w1s, w1d, b1, w2, b2, w3, b3,
                    lns, lnb, o_ref):
    h = jnp.maximum(jnp.dot(e_ref[...], w1[...],
                            preferred_element_type=jnp.float32)
                    + jnp.dot(gs_ref[...], w1s[...],
                              preferred_element_type=jnp.float32)
                    + jnp.dot(gd_ref[...], w1d[...],
                              preferred_element_type=jnp.float32)
                    + b1[...], 0.0)
    h = jnp.maximum(jnp.dot(h, w2[...], preferred_element_type=jnp.float32,
                           ) + b2[...], 0.0)
    h = jnp.dot(h, w3[...], preferred_element_type=jnp.float32,
               ) + b3[...]
    o_ref[...] = _ln(h, lns[...], lnb[...]) + e_ref[...]


def _node_proc_body(x_ref, p0_ref, p1_ref, w1x, w1a, b1, w2, b2, w3, b3,
                    lns, lnb, o_ref):
    agg = p0_ref[...] + p1_ref[...]
    h = jnp.maximum(jnp.dot(x_ref[...], w1x[...],
                            preferred_element_type=jnp.float32)
                    + jnp.dot(agg, w1a[...],
                              preferred_element_type=jnp.float32,
                             )
                    + b1[...], 0.0)
    h = jnp.maximum(jnp.dot(h, w2[...], preferred_element_type=jnp.float32,
                           ) + b2[...], 0.0)
    h = jnp.dot(h, w3[...], preferred_element_type=jnp.float32,
               ) + b3[...]
    o_ref[...] = _ln(h, lns[...], lnb[...]) + x_ref[...]


def _row_call(body, n_rows, blk, row_args, full_args, out_dim):
    grid = (n_rows // blk,)
    in_specs = []
    for a in row_args:
        in_specs.append(pl.BlockSpec((blk, a.shape[1]), lambda i: (i, 0)))
    for a in full_args:
        in_specs.append(
            pl.BlockSpec(a.shape, lambda i: tuple(0 for _ in a.shape)))
    out_shape = jax.ShapeDtypeStruct((n_rows, out_dim), jnp.float32)
    out_spec = pl.BlockSpec((blk, out_dim), lambda i: (i, 0))
    return pl.pallas_call(
        body, grid=grid, in_specs=in_specs, out_specs=out_spec,
        out_shape=out_shape, interpret=INTERPRET,
    )(*row_args, *full_args)


def _make_sc_gather():
    mesh = plsc.VectorSubcoreMesh(core_axis_name="c", subcore_axis_name="s")
    gshape = jax.ShapeDtypeStruct((NE_PAD, 128), jnp.float32)

    GROUPS = NCHUNK // 2

    @functools.partial(
        pl.kernel,
        out_type=[gshape, gshape],
        mesh=mesh,
        scratch_types=[
            pltpu.VMEM((NCHUNK, K), jnp.int32),
            pltpu.VMEM((NCHUNK, K), jnp.int32),
            pltpu.VMEM((K, 128), jnp.float32),
            pltpu.VMEM((K, 128), jnp.float32),
            pltpu.VMEM((K, 128), jnp.float32),
            pltpu.VMEM((K, 128), jnp.float32),
            pltpu.SemaphoreType.DMA,
            pltpu.SemaphoreType.DMA,
            pltpu.SemaphoreType.DMA,
            pltpu.SemaphoreType.DMA,
            pltpu.SemaphoreType.DMA,
            pltpu.SemaphoreType.DMA,
            pltpu.SemaphoreType.DMA,
            pltpu.SemaphoreType.DMA,
        ])
    def gather_k(xs_hbm, xd_hbm, src_hbm, dst_hbm, gs_hbm, gd_hbm,
                 src_v, dst_v, a0, b0, a1, b1,
                 sa0, sb0, sa1, sb1, oa0, ob0, oa1, ob1):
        cid = lax.axis_index("c")
        sid = lax.axis_index("s")
        wid = sid * 2 + cid
        pltpu.sync_copy(src_hbm.at[wid], src_v)
        pltpu.sync_copy(dst_hbm.at[wid], dst_v)
        base = wid * C

        def fire_g(j, ba, bb, sa, sb):
            pltpu.async_copy(xs_hbm.at[src_v.at[j]], ba, sa)
            pltpu.async_copy(xd_hbm.at[dst_v.at[j]], bb, sb)

        def wait_g(j, ba, bb, sa, sb):
            pltpu.make_async_copy(xs_hbm.at[src_v.at[j]], ba, sa).wait()
            pltpu.make_async_copy(xd_hbm.at[dst_v.at[j]], bb, sb).wait()

        def fire_o(j, ba, bb, oa, ob):
            rows = pl.ds(base + j * K, K)
            pltpu.async_copy(ba, gs_hbm.at[rows], oa)
            pltpu.async_copy(bb, gd_hbm.at[rows], ob)

        def wait_o(j, ba, bb, oa, ob):
            rows = pl.ds(base + j * K, K)
            pltpu.make_async_copy(ba, gs_hbm.at[rows], oa).wait()
            pltpu.make_async_copy(bb, gd_hbm.at[rows], ob).wait()

        fire_g(0, a0, b0, sa0, sb0)

        def gbody(g, _):
            j0 = 2 * g
            j1 = j0 + 1
            fire_g(j1, a1, b1, sa1, sb1)
            wait_g(j0, a0, b0, sa0, sb0)
            fire_o(j0, a0, b0, oa0, ob0)
            wait_g(j1, a1, b1, sa1, sb1)
            fire_o(j1, a1, b1, oa1, ob1)
            wait_o(j0, a0, b0, oa0, ob0)

            @pl.when(g < GROUPS - 1)
            def _():
                fire_g(j0 + 2, a0, b0, sa0, sb0)

            wait_o(j1, a1, b1, oa1, ob1)
            return 0

        lax.fori_loop(0, GROUPS, gbody, 0)

    return gather_k


def _make_sc_scatter(n_nodes, n_acc):
    mesh = plsc.VectorSubcoreMesh(core_axis_name="c", subcore_axis_name="s")
    rows_per_tile = n_acc // 16
    nz_full, nz_rem = divmod(rows_per_tile, K)

    @functools.partial(
        pl.kernel,
        out_type=jax.ShapeDtypeStruct((2, n_acc, 128), jnp.float32),
        mesh=mesh,
        scratch_types=[
            pltpu.VMEM((NCHUNK, K), jnp.int32),
            pltpu.VMEM((K, 128), jnp.float32),
            pltpu.VMEM((K, 128), jnp.float32),
            pltpu.VMEM_SHARED((n_acc, 128), jnp.float32),
            pltpu.SemaphoreType.DMA,
            pltpu.SemaphoreType.DMA,
        ])
    def scatter_k(e_hbm, dst_hbm, out_hbm, dst_v, e0, e1, acc, s0, s1):
        cid = lax.axis_index("c")
        sid = lax.axis_index("s")
        wid = sid * 2 + cid
        pltpu.sync_copy(dst_hbm.at[wid], dst_v)

        def zrow(r, _):
            for c2 in range(8):
                e0[r, pl.ds(c2 * 16, 16)] = jnp.zeros((16,), jnp.float32)
            return 0

        lax.fori_loop(0, K, zrow, 0)
        for t in range(nz_full):
            pltpu.sync_copy(
                e0, acc.at[pl.ds(sid * rows_per_tile + t * K, K)])
        if nz_rem:
            pltpu.sync_copy(
                e0.at[pl.ds(0, nz_rem)],
                acc.at[pl.ds(sid * rows_per_tile + nz_full * K, nz_rem)])
        plsc.subcore_barrier()

        def fire_e(j, eb, sem):
            pltpu.async_copy(
                e_hbm.at[pl.ds(wid * C + j * K, K)], eb, sem)

        def wait_e(j, eb, sem):
            pltpu.make_async_copy(
                e_hbm.at[pl.ds(wid * C + j * K, K)], eb, sem).wait()

        fire_e(0, e0, s0)

        def sbody(g, _):
            j0 = 2 * g
            j1 = j0 + 1
            fire_e(j1, e1, s1)
            wait_e(j0, e0, s0)
            pltpu.sync_copy(e0, acc.at[dst_v.at[j0]], add=True)

            @pl.when(g < NCHUNK // 2 - 1)
            def _():
                fire_e(j0 + 2, e0, s0)

            wait_e(j1, e1, s1)
            pltpu.sync_copy(e1, acc.at[dst_v.at[j1]], add=True)
            return 0

        lax.fori_loop(0, NCHUNK // 2, sbody, 0)
        plsc.subcore_barrier()
        rows = pl.ds(sid * rows_per_tile, rows_per_tile)
        pltpu.sync_copy(acc.at[rows], out_hbm.at[cid, rows])

    return scatter_k


def _mlp_params(p):
    out = []
    for W, b in zip(p["Ws"], p["bs"]):
        out.append(W)
        out.append(b.reshape(1, -1))
    if "ln_scale" in p:
        out.append(p["ln_scale"].reshape(1, -1))
        out.append(p["ln_bias"].reshape(1, -1))
    return out


def kernel(node_features, edge_features, edge_index, params):
    n_nodes = node_features.shape[0]
    n_edges = edge_features.shape[0]
    n_acc = (n_nodes + 128) // 128 * 128
    src = edge_index[0]
    dst = edge_index[1]

    pad = NE_PAD - n_edges
    zpad = jnp.zeros((pad,), jnp.int32)
    src_g = jnp.concatenate([src, zpad]).reshape(NW, NCHUNK, K)
    dst_g = jnp.concatenate([dst, zpad]).reshape(NW, NCHUNK, K)
    dst_s = jnp.concatenate(
        [dst, jnp.full((pad,), n_nodes, jnp.int32)]).reshape(NW, NCHUNK, K)
    ef_pad = jnp.concatenate(
        [edge_features, jnp.zeros((pad, edge_features.shape[1]),
                                  jnp.float32)], axis=0)

    sc_gather = _make_sc_gather()
    sc_scatter = _make_sc_scatter(n_nodes, n_acc)

    e = _row_call(_enc_body, NE_PAD, N_EDGE_BLK, [ef_pad],
                  _mlp_params(params["edge_enc"]), 128)
    x = _row_call(_enc_body, n_nodes, N_NODE_BLK, [node_features],
                  _mlp_params(params["node_enc"]), 128)

    for layer in params["processor"]:
        ep = layer["edge_mlp"]
        npm = layer["node_mlp"]
        W1 = ep["Ws"][0]
        gs, gd = sc_gather(x, x, src_g, dst_g)
        ep_rest = [W1[:128], W1[128:256], W1[256:384],
                   ep["bs"][0].reshape(1, -1),
                   ep["Ws"][1], ep["bs"][1].reshape(1, -1),
                   ep["Ws"][2], ep["bs"][2].reshape(1, -1),
                   ep["ln_scale"].reshape(1, -1),
                   ep["ln_bias"].reshape(1, -1)]
        e = _row_call(_edge_proc_body, NE_PAD, N_EDGE_BLK, [e, gs, gd],
                      ep_rest, 128)
        part = sc_scatter(e, dst_s)
        nW1 = npm["Ws"][0]
        np_rest = [nW1[:128], nW1[128:], npm["bs"][0].reshape(1, -1),
                   npm["Ws"][1], npm["bs"][1].reshape(1, -1),
                   npm["Ws"][2], npm["bs"][2].reshape(1, -1),
                   npm["ln_scale"].reshape(1, -1),
                   npm["ln_bias"].reshape(1, -1)]
        x = _row_call(_node_proc_body, n_nodes, N_NODE_BLK,
                      [x, part[0], part[1]], np_rest, 128)

    out = _row_call(_dec_body, n_nodes, N_NODE_BLK, [x],
                    _mlp_params(params["node_dec"]), 3)
    return out


# speedup vs baseline: 2.3730x; 1.1234x over previous
import functools

import jax
import jax.numpy as jnp
from jax import lax
from jax.experimental import pallas as pl
from jax.experimental.pallas import tpu as pltpu
from jax.experimental.pallas import tpu_sc as plsc

INTERPRET = False

NW = 32
K = 128
NCHUNK = 80
C = NCHUNK * K
NE_PAD = NW * C

N_EDGE_BLK = 2560
N_NODE_BLK = 2000

HI = jax.lax.Precision.HIGHEST


def _ln(x, scale, bias):
    mu = jnp.mean(x, axis=-1, keepdims=True)
    var = jnp.mean((x - mu) ** 2, axis=-1, keepdims=True)
    return (x - mu) * jax.lax.rsqrt(var + 1e-5) * scale + bias


def _enc_body(x_ref, w1, b1, w2, b2, w3, b3, lns, lnb, o_ref):
    h = jnp.maximum(jnp.dot(x_ref[...], w1[...],
                            preferred_element_type=jnp.float32,
                           ) + b1[...], 0.0)
    h = jnp.maximum(jnp.dot(h, w2[...], preferred_element_type=jnp.float32,
                           ) + b2[...], 0.0)
    h = jnp.dot(h, w3[...], preferred_element_type=jnp.float32,
               ) + b3[...]
    o_ref[...] = _ln(h, lns[...], lnb[...])


def _dec_body(x_ref, w1, b1, w2, b2, w3, b3, o_ref):
    h = jnp.maximum(jnp.dot(x_ref[...], w1[...],
                            preferred_element_type=jnp.float32,
                           ) + b1[...], 0.0)
    h = jnp.maximum(jnp.dot(h, w2[...], preferred_element_type=jnp.float32,
                           ) + b2[...], 0.0)
    o_ref[...] = jnp.dot(h, w3[...], preferred_element_type=jnp.float32,
                        ) + b3[...]


def _edge_proc_body(e_ref, gs_ref, gd_ref, w1, w1s, w1d, b1, w2, b2, w3, b3,
                    lns, lnb, o_ref):
    h = jnp.maximum(jnp.dot(e_ref[...], w1[...],
                            preferred_element_type=jnp.float32)
                    + jnp.dot(gs_ref[...], w1s[...],
                              preferred_element_type=jnp.float32)
                    + jnp.dot(gd_ref[...], w1d[...],
                              preferred_element_type=jnp.float32)
                    + b1[...], 0.0)
    h = jnp.maximum(jnp.dot(h, w2[...], preferred_element_type=jnp.float32,
                           ) + b2[...], 0.0)
    h = jnp.dot(h, w3[...], preferred_element_type=jnp.float32,
               ) + b3[...]
    o_ref[...] = _ln(h, lns[...], lnb[...]) + e_ref[...]


def _node_proc_body(x_ref, p0_ref, p1_ref, p2_ref, p3_ref, w1x, w1a, b1,
                    w2, b2, w3, b3, lns, lnb, o_ref):
    agg = (p0_ref[...] + p1_ref[...]) + (p2_ref[...] + p3_ref[...])
    h = jnp.maximum(jnp.dot(x_ref[...], w1x[...],
                            preferred_element_type=jnp.float32)
                    + jnp.dot(agg, w1a[...],
                              preferred_element_type=jnp.float32,
                             )
                    + b1[...], 0.0)
    h = jnp.maximum(jnp.dot(h, w2[...], preferred_element_type=jnp.float32,
                           ) + b2[...], 0.0)
    h = jnp.dot(h, w3[...], preferred_element_type=jnp.float32,
               ) + b3[...]
    o_ref[...] = _ln(h, lns[...], lnb[...]) + x_ref[...]


def _row_call(body, n_rows, blk, row_args, full_args, out_dim):
    grid = (n_rows // blk,)
    in_specs = []
    for a in row_args:
        in_specs.append(pl.BlockSpec((blk, a.shape[1]), lambda i: (i, 0)))
    for a in full_args:
        in_specs.append(
            pl.BlockSpec(a.shape, lambda i: tuple(0 for _ in a.shape)))
    out_shape = jax.ShapeDtypeStruct((n_rows, out_dim), jnp.float32)
    out_spec = pl.BlockSpec((blk, out_dim), lambda i: (i, 0))
    return pl.pallas_call(
        body, grid=grid, in_specs=in_specs, out_specs=out_spec,
        out_shape=out_shape, interpret=INTERPRET,
    )(*row_args, *full_args)


def _make_sc_gather(nchunk, c):
    mesh = plsc.VectorSubcoreMesh(core_axis_name="c", subcore_axis_name="s")
    gshape = jax.ShapeDtypeStruct((NW * c, 128), jnp.float32)

    GROUPS = nchunk // 2

    @functools.partial(
        pl.kernel,
        out_type=[gshape, gshape],
        mesh=mesh,
        scratch_types=[
            pltpu.VMEM((nchunk, K), jnp.int32),
            pltpu.VMEM((nchunk, K), jnp.int32),
            pltpu.VMEM((K, 128), jnp.float32),
            pltpu.VMEM((K, 128), jnp.float32),
            pltpu.VMEM((K, 128), jnp.float32),
            pltpu.VMEM((K, 128), jnp.float32),
            pltpu.SemaphoreType.DMA,
            pltpu.SemaphoreType.DMA,
            pltpu.SemaphoreType.DMA,
            pltpu.SemaphoreType.DMA,
            pltpu.SemaphoreType.DMA,
            pltpu.SemaphoreType.DMA,
            pltpu.SemaphoreType.DMA,
            pltpu.SemaphoreType.DMA,
        ])
    def gather_k(xs_hbm, xd_hbm, src_hbm, dst_hbm, gs_hbm, gd_hbm,
                 src_v, dst_v, a0, b0, a1, b1,
                 sa0, sb0, sa1, sb1, oa0, ob0, oa1, ob1):
        cid = lax.axis_index("c")
        sid = lax.axis_index("s")
        wid = sid * 2 + cid
        pltpu.sync_copy(src_hbm.at[wid], src_v)
        pltpu.sync_copy(dst_hbm.at[wid], dst_v)
        base = wid * c

        def fire_g(j, ba, bb, sa, sb):
            pltpu.async_copy(xs_hbm.at[src_v.at[j]], ba, sa)
            pltpu.async_copy(xd_hbm.at[dst_v.at[j]], bb, sb)

        def wait_g(j, ba, bb, sa, sb):
            pltpu.make_async_copy(xs_hbm.at[src_v.at[j]], ba, sa).wait()
            pltpu.make_async_copy(xd_hbm.at[dst_v.at[j]], bb, sb).wait()

        def fire_o(j, ba, bb, oa, ob):
            rows = pl.ds(base + j * K, K)
            pltpu.async_copy(ba, gs_hbm.at[rows], oa)
            pltpu.async_copy(bb, gd_hbm.at[rows], ob)

        def wait_o(j, ba, bb, oa, ob):
            rows = pl.ds(base + j * K, K)
            pltpu.make_async_copy(ba, gs_hbm.at[rows], oa).wait()
            pltpu.make_async_copy(bb, gd_hbm.at[rows], ob).wait()

        fire_g(0, a0, b0, sa0, sb0)

        def gbody(g, _):
            j0 = 2 * g
            j1 = j0 + 1
            fire_g(j1, a1, b1, sa1, sb1)
            wait_g(j0, a0, b0, sa0, sb0)
            fire_o(j0, a0, b0, oa0, ob0)
            wait_g(j1, a1, b1, sa1, sb1)
            fire_o(j1, a1, b1, oa1, ob1)
            wait_o(j0, a0, b0, oa0, ob0)

            @pl.when(g < GROUPS - 1)
            def _():
                fire_g(j0 + 2, a0, b0, sa0, sb0)

            wait_o(j1, a1, b1, oa1, ob1)
            return 0

        lax.fori_loop(0, GROUPS, gbody, 0)

    return gather_k


def _make_sc_scatter(n_nodes, n_acc, nchunk, c):
    mesh = plsc.VectorSubcoreMesh(core_axis_name="c", subcore_axis_name="s")
    rows_per_tile = n_acc // 16
    nz_full, nz_rem = divmod(rows_per_tile, K)

    @functools.partial(
        pl.kernel,
        out_type=jax.ShapeDtypeStruct((2, n_acc, 128), jnp.float32),
        mesh=mesh,
        scratch_types=[
            pltpu.VMEM((nchunk, K), jnp.int32),
            pltpu.VMEM((K, 128), jnp.float32),
            pltpu.VMEM((K, 128), jnp.float32),
            pltpu.VMEM_SHARED((n_acc, 128), jnp.float32),
            pltpu.SemaphoreType.DMA,
            pltpu.SemaphoreType.DMA,
        ])
    def scatter_k(e_hbm, dst_hbm, out_hbm, dst_v, e0, e1, acc, s0, s1):
        cid = lax.axis_index("c")
        sid = lax.axis_index("s")
        wid = sid * 2 + cid
        pltpu.sync_copy(dst_hbm.at[wid], dst_v)

        def zrow(r, _):
            for c2 in range(8):
                e0[r, pl.ds(c2 * 16, 16)] = jnp.zeros((16,), jnp.float32)
            return 0

        lax.fori_loop(0, K, zrow, 0)
        for t in range(nz_full):
            pltpu.sync_copy(
                e0, acc.at[pl.ds(sid * rows_per_tile + t * K, K)])
        if nz_rem:
            pltpu.sync_copy(
                e0.at[pl.ds(0, nz_rem)],
                acc.at[pl.ds(sid * rows_per_tile + nz_full * K, nz_rem)])
        plsc.subcore_barrier()

        def fire_e(j, eb, sem):
            pltpu.async_copy(
                e_hbm.at[pl.ds(wid * c + j * K, K)], eb, sem)

        def wait_e(j, eb, sem):
            pltpu.make_async_copy(
                e_hbm.at[pl.ds(wid * c + j * K, K)], eb, sem).wait()

        fire_e(0, e0, s0)

        def sbody(g, _):
            j0 = 2 * g
            j1 = j0 + 1
            fire_e(j1, e1, s1)
            wait_e(j0, e0, s0)
            pltpu.sync_copy(e0, acc.at[dst_v.at[j0]], add=True)

            @pl.when(g < nchunk // 2 - 1)
            def _():
                fire_e(j0 + 2, e0, s0)

            wait_e(j1, e1, s1)
            pltpu.sync_copy(e1, acc.at[dst_v.at[j1]], add=True)
            return 0

        lax.fori_loop(0, nchunk // 2, sbody, 0)
        plsc.subcore_barrier()
        rows = pl.ds(sid * rows_per_tile, rows_per_tile)
        pltpu.sync_copy(acc.at[rows], out_hbm.at[cid, rows])

    return scatter_k


def _mlp_params(p):
    out = []
    for W, b in zip(p["Ws"], p["bs"]):
        out.append(W)
        out.append(b.reshape(1, -1))
    if "ln_scale" in p:
        out.append(p["ln_scale"].reshape(1, -1))
        out.append(p["ln_bias"].reshape(1, -1))
    return out


def kernel(node_features, edge_features, edge_index, params):
    n_nodes = node_features.shape[0]
    n_edges = edge_features.shape[0]
    n_acc = (n_nodes + 128) // 128 * 128
    src = edge_index[0]
    dst = edge_index[1]

    NH = 2
    NE_H = NE_PAD // NH
    NCHUNK_H = NCHUNK // NH
    C_H = NCHUNK_H * K
    pad = NE_PAD - n_edges
    zpad = jnp.zeros((pad,), jnp.int32)
    src_p = jnp.concatenate([src, zpad])
    dst_p = jnp.concatenate([dst, zpad])
    dst_sp = jnp.concatenate([dst, jnp.full((pad,), n_nodes, jnp.int32)])
    src_g = [src_p[h * NE_H:(h + 1) * NE_H].reshape(NW, NCHUNK_H, K)
             for h in range(NH)]
    dst_g = [dst_p[h * NE_H:(h + 1) * NE_H].reshape(NW, NCHUNK_H, K)
             for h in range(NH)]
    dst_s = [dst_sp[h * NE_H:(h + 1) * NE_H].reshape(NW, NCHUNK_H, K)
             for h in range(NH)]
    ef_pad = jnp.concatenate(
        [edge_features, jnp.zeros((pad, edge_features.shape[1]),
                                  jnp.float32)], axis=0)

    sc_gather = _make_sc_gather(NCHUNK_H, C_H)
    sc_scatter = _make_sc_scatter(n_nodes, n_acc, NCHUNK_H, C_H)

    e = [_row_call(_enc_body, NE_H, N_EDGE_BLK,
                   [ef_pad[h * NE_H:(h + 1) * NE_H]],
                   _mlp_params(params["edge_enc"]), 128)
         for h in range(NH)]
    x = _row_call(_enc_body, n_nodes, N_NODE_BLK, [node_features],
                  _mlp_params(params["node_enc"]), 128)

    for layer in params["processor"]:
        ep = layer["edge_mlp"]
        npm = layer["node_mlp"]
        W1 = ep["Ws"][0]
        ep_rest = [W1[:128], W1[128:256], W1[256:384],
                   ep["bs"][0].reshape(1, -1),
                   ep["Ws"][1], ep["bs"][1].reshape(1, -1),
                   ep["Ws"][2], ep["bs"][2].reshape(1, -1),
                   ep["ln_scale"].reshape(1, -1),
                   ep["ln_bias"].reshape(1, -1)]
        g = [sc_gather(x, x, src_g[h], dst_g[h]) for h in range(NH)]
        e = [_row_call(_edge_proc_body, NE_H, N_EDGE_BLK,
                       [e[h], g[h][0], g[h][1]], ep_rest, 128)
             for h in range(NH)]
        part = [sc_scatter(e[h], dst_s[h]) for h in range(NH)]
        nW1 = npm["Ws"][0]
        np_rest = [nW1[:128], nW1[128:], npm["bs"][0].reshape(1, -1),
                   npm["Ws"][1], npm["bs"][1].reshape(1, -1),
                   npm["Ws"][2], npm["bs"][2].reshape(1, -1),
                   npm["ln_scale"].reshape(1, -1),
                   npm["ln_bias"].reshape(1, -1)]
        x = _row_call(_node_proc_body, n_nodes, N_NODE_BLK,
                      [x, part[0][0], part[0][1], part[1][0], part[1][1]],
                      np_rest, 128)

    out = _row_call(_dec_body, n_nodes, N_NODE_BLK, [x],
                    _mlp_params(params["node_dec"]), 3)
    return out


# speedup vs baseline: 2.3754x; 1.0010x over previous
import functools

import jax
import jax.numpy as jnp
from jax import lax
from jax.experimental import pallas as pl
from jax.experimental.pallas import tpu as pltpu
from jax.experimental.pallas import tpu_sc as plsc

INTERPRET = False

NW = 32
K = 128
NCHUNK = 80
C = NCHUNK * K
NE_PAD = NW * C

N_EDGE_BLK = 2560
N_NODE_BLK = 2000

HI = jax.lax.Precision.HIGHEST


def _ln(x, scale, bias):
    mu = jnp.mean(x, axis=-1, keepdims=True)
    var = jnp.mean((x - mu) ** 2, axis=-1, keepdims=True)
    return (x - mu) * jax.lax.rsqrt(var + 1e-5) * scale + bias


def _enc_body(x_ref, w1, b1, w2, b2, w3, b3, lns, lnb, o_ref):
    h = jnp.maximum(jnp.dot(x_ref[...], w1[...],
                            preferred_element_type=jnp.float32,
                           ) + b1[...], 0.0)
    h = jnp.maximum(jnp.dot(h, w2[...], preferred_element_type=jnp.float32,
                           ) + b2[...], 0.0)
    h = jnp.dot(h, w3[...], preferred_element_type=jnp.float32,
               ) + b3[...]
    o_ref[...] = _ln(h, lns[...], lnb[...])


def _dec_body(x_ref, w1, b1, w2, b2, w3, b3, o_ref):
    h = jnp.maximum(jnp.dot(x_ref[...], w1[...],
                            preferred_element_type=jnp.float32,
                           ) + b1[...], 0.0)
    h = jnp.maximum(jnp.dot(h, w2[...], preferred_element_type=jnp.float32,
                           ) + b2[...], 0.0)
    o_ref[...] = jnp.dot(h, w3[...], preferred_element_type=jnp.float32,
                        ) + b3[...]


def _edge_proc_body(e_ref, gs_ref, gd_ref, w1, w1s, w1d, b1, w2, b2, w3, b3,
                    lns, lnb, o_ref):
    h = jnp.maximum(jnp.dot(e_ref[...], w1[...],
                            preferred_element_type=jnp.float32)
                    + jnp.dot(gs_ref[...], w1s[...],
                              preferred_element_type=jnp.float32)
                    + jnp.dot(gd_ref[...], w1d[...],
                              preferred_element_type=jnp.float32)
                    + b1[...], 0.0)
    h = jnp.maximum(jnp.dot(h, w2[...], preferred_element_type=jnp.float32,
                           ) + b2[...], 0.0)
    h = jnp.dot(h, w3[...], preferred_element_type=jnp.float32,
               ) + b3[...]
    o_ref[...] = _ln(h, lns[...], lnb[...]) + e_ref[...]


def _make_node_body(nparts):
    def body(x_ref, *args):
        parts = args[:nparts]
        w1x, w1a, b1, w2, b2, w3, b3, lns, lnb, o_ref = args[nparts:]
        agg = parts[0][...]
        for p in parts[1:]:
            agg = agg + p[...]
        h = jnp.maximum(jnp.dot(x_ref[...], w1x[...],
                                preferred_element_type=jnp.float32)
                        + jnp.dot(agg, w1a[...],
                                  preferred_element_type=jnp.float32)
                        + b1[...], 0.0)
        h = jnp.maximum(jnp.dot(h, w2[...],
                                preferred_element_type=jnp.float32)
                        + b2[...], 0.0)
        h = jnp.dot(h, w3[...], preferred_element_type=jnp.float32) + b3[...]
        o_ref[...] = _ln(h, lns[...], lnb[...]) + x_ref[...]
    return body


def _row_call(body, n_rows, blk, row_args, full_args, out_dim):
    grid = (n_rows // blk,)
    in_specs = []
    for a in row_args:
        in_specs.append(pl.BlockSpec((blk, a.shape[1]), lambda i: (i, 0)))
    for a in full_args:
        in_specs.append(
            pl.BlockSpec(a.shape, lambda i: tuple(0 for _ in a.shape)))
    out_shape = jax.ShapeDtypeStruct((n_rows, out_dim), jnp.float32)
    out_spec = pl.BlockSpec((blk, out_dim), lambda i: (i, 0))
    return pl.pallas_call(
        body, grid=grid, in_specs=in_specs, out_specs=out_spec,
        out_shape=out_shape, interpret=INTERPRET,
    )(*row_args, *full_args)


def _make_sc_gather(nchunk, c):
    mesh = plsc.VectorSubcoreMesh(core_axis_name="c", subcore_axis_name="s")
    gshape = jax.ShapeDtypeStruct((NW * c, 128), jnp.float32)

    GROUPS = nchunk // 2

    @functools.partial(
        pl.kernel,
        out_type=[gshape, gshape],
        mesh=mesh,
        scratch_types=[
            pltpu.VMEM((nchunk, K), jnp.int32),
            pltpu.VMEM((nchunk, K), jnp.int32),
            pltpu.VMEM((K, 128), jnp.float32),
            pltpu.VMEM((K, 128), jnp.float32),
            pltpu.VMEM((K, 128), jnp.float32),
            pltpu.VMEM((K, 128), jnp.float32),
            pltpu.SemaphoreType.DMA,
            pltpu.SemaphoreType.DMA,
            pltpu.SemaphoreType.DMA,
            pltpu.SemaphoreType.DMA,
            pltpu.SemaphoreType.DMA,
            pltpu.SemaphoreType.DMA,
            pltpu.SemaphoreType.DMA,
            pltpu.SemaphoreType.DMA,
        ])
    def gather_k(xs_hbm, xd_hbm, src_hbm, dst_hbm, gs_hbm, gd_hbm,
                 src_v, dst_v, a0, b0, a1, b1,
                 sa0, sb0, sa1, sb1, oa0, ob0, oa1, ob1):
        cid = lax.axis_index("c")
        sid = lax.axis_index("s")
        wid = sid * 2 + cid
        pltpu.sync_copy(src_hbm.at[wid], src_v)
        pltpu.sync_copy(dst_hbm.at[wid], dst_v)
        base = wid * c

        def fire_g(j, ba, bb, sa, sb):
            pltpu.async_copy(xs_hbm.at[src_v.at[j]], ba, sa)
            pltpu.async_copy(xd_hbm.at[dst_v.at[j]], bb, sb)

        def wait_g(j, ba, bb, sa, sb):
            pltpu.make_async_copy(xs_hbm.at[src_v.at[j]], ba, sa).wait()
            pltpu.make_async_copy(xd_hbm.at[dst_v.at[j]], bb, sb).wait()

        def fire_o(j, ba, bb, oa, ob):
            rows = pl.ds(base + j * K, K)
            pltpu.async_copy(ba, gs_hbm.at[rows], oa)
            pltpu.async_copy(bb, gd_hbm.at[rows], ob)

        def wait_o(j, ba, bb, oa, ob):
            rows = pl.ds(base + j * K, K)
            pltpu.make_async_copy(ba, gs_hbm.at[rows], oa).wait()
            pltpu.make_async_copy(bb, gd_hbm.at[rows], ob).wait()

        fire_g(0, a0, b0, sa0, sb0)

        def gbody(g, _):
            j0 = 2 * g
            j1 = j0 + 1
            fire_g(j1, a1, b1, sa1, sb1)
            wait_g(j0, a0, b0, sa0, sb0)
            fire_o(j0, a0, b0, oa0, ob0)
            wait_g(j1, a1, b1, sa1, sb1)
            fire_o(j1, a1, b1, oa1, ob1)
            wait_o(j0, a0, b0, oa0, ob0)

            @pl.when(g < GROUPS - 1)
            def _():
                fire_g(j0 + 2, a0, b0, sa0, sb0)

            wait_o(j1, a1, b1, oa1, ob1)
            return 0

        lax.fori_loop(0, GROUPS, gbody, 0)

    return gather_k


def _make_sc_scatter(n_nodes, n_acc, nchunk, c):
    mesh = plsc.VectorSubcoreMesh(core_axis_name="c", subcore_axis_name="s")
    rows_per_tile = n_acc // 16
    nz_full, nz_rem = divmod(rows_per_tile, K)

    @functools.partial(
        pl.kernel,
        out_type=jax.ShapeDtypeStruct((2, n_acc, 128), jnp.float32),
        mesh=mesh,
        scratch_types=[
            pltpu.VMEM((nchunk, K), jnp.int32),
            pltpu.VMEM((K, 128), jnp.float32),
            pltpu.VMEM((K, 128), jnp.float32),
            pltpu.VMEM_SHARED((n_acc, 128), jnp.float32),
            pltpu.SemaphoreType.DMA,
            pltpu.SemaphoreType.DMA,
        ])
    def scatter_k(e_hbm, dst_hbm, out_hbm, dst_v, e0, e1, acc, s0, s1):
        cid = lax.axis_index("c")
        sid = lax.axis_index("s")
        wid = sid * 2 + cid
        pltpu.sync_copy(dst_hbm.at[wid], dst_v)

        def zrow(r, _):
            for c2 in range(8):
                e0[r, pl.ds(c2 * 16, 16)] = jnp.zeros((16,), jnp.float32)
            return 0

        lax.fori_loop(0, K, zrow, 0)
        for t in range(nz_full):
            pltpu.sync_copy(
                e0, acc.at[pl.ds(sid * rows_per_tile + t * K, K)])
        if nz_rem:
            pltpu.sync_copy(
                e0.at[pl.ds(0, nz_rem)],
                acc.at[pl.ds(sid * rows_per_tile + nz_full * K, nz_rem)])
        plsc.subcore_barrier()

        def fire_e(j, eb, sem):
            pltpu.async_copy(
                e_hbm.at[pl.ds(wid * c + j * K, K)], eb, sem)

        def wait_e(j, eb, sem):
            pltpu.make_async_copy(
                e_hbm.at[pl.ds(wid * c + j * K, K)], eb, sem).wait()

        fire_e(0, e0, s0)

        def sbody(g, _):
            j0 = 2 * g
            j1 = j0 + 1
            fire_e(j1, e1, s1)
            wait_e(j0, e0, s0)
            pltpu.sync_copy(e0, acc.at[dst_v.at[j0]], add=True)

            @pl.when(g < nchunk // 2 - 1)
            def _():
                fire_e(j0 + 2, e0, s0)

            wait_e(j1, e1, s1)
            pltpu.sync_copy(e1, acc.at[dst_v.at[j1]], add=True)
            return 0

        lax.fori_loop(0, nchunk // 2, sbody, 0)
        plsc.subcore_barrier()
        rows = pl.ds(sid * rows_per_tile, rows_per_tile)
        pltpu.sync_copy(acc.at[rows], out_hbm.at[cid, rows])

    return scatter_k


def _mlp_params(p):
    out = []
    for W, b in zip(p["Ws"], p["bs"]):
        out.append(W)
        out.append(b.reshape(1, -1))
    if "ln_scale" in p:
        out.append(p["ln_scale"].reshape(1, -1))
        out.append(p["ln_bias"].reshape(1, -1))
    return out


def kernel(node_features, edge_features, edge_index, params):
    n_nodes = node_features.shape[0]
    n_edges = edge_features.shape[0]
    n_acc = (n_nodes + 128) // 128 * 128
    src = edge_index[0]
    dst = edge_index[1]

    NH = 4
    NE_H = NE_PAD // NH
    NCHUNK_H = NCHUNK // NH
    C_H = NCHUNK_H * K
    pad = NE_PAD - n_edges
    zpad = jnp.zeros((pad,), jnp.int32)
    src_p = jnp.concatenate([src, zpad])
    dst_p = jnp.concatenate([dst, zpad])
    dst_sp = jnp.concatenate([dst, jnp.full((pad,), n_nodes, jnp.int32)])
    src_g = [src_p[h * NE_H:(h + 1) * NE_H].reshape(NW, NCHUNK_H, K)
             for h in range(NH)]
    dst_g = [dst_p[h * NE_H:(h + 1) * NE_H].reshape(NW, NCHUNK_H, K)
             for h in range(NH)]
    dst_s = [dst_sp[h * NE_H:(h + 1) * NE_H].reshape(NW, NCHUNK_H, K)
             for h in range(NH)]
    ef_pad = jnp.concatenate(
        [edge_features, jnp.zeros((pad, edge_features.shape[1]),
                                  jnp.float32)], axis=0)

    sc_gather = _make_sc_gather(NCHUNK_H, C_H)
    sc_scatter = _make_sc_scatter(n_nodes, n_acc, NCHUNK_H, C_H)

    e = [_row_call(_enc_body, NE_H, N_EDGE_BLK,
                   [ef_pad[h * NE_H:(h + 1) * NE_H]],
                   _mlp_params(params["edge_enc"]), 128)
         for h in range(NH)]
    x = _row_call(_enc_body, n_nodes, N_NODE_BLK, [node_features],
                  _mlp_params(params["node_enc"]), 128)

    for layer in params["processor"]:
        ep = layer["edge_mlp"]
        npm = layer["node_mlp"]
        W1 = ep["Ws"][0]
        ep_rest = [W1[:128], W1[128:256], W1[256:384],
                   ep["bs"][0].reshape(1, -1),
                   ep["Ws"][1], ep["bs"][1].reshape(1, -1),
                   ep["Ws"][2], ep["bs"][2].reshape(1, -1),
                   ep["ln_scale"].reshape(1, -1),
                   ep["ln_bias"].reshape(1, -1)]
        g = [sc_gather(x, x, src_g[h], dst_g[h]) for h in range(NH)]
        e = [_row_call(_edge_proc_body, NE_H, N_EDGE_BLK,
                       [e[h], g[h][0], g[h][1]], ep_rest, 128)
             for h in range(NH)]
        part = [sc_scatter(e[h], dst_s[h]) for h in range(NH)]
        nW1 = npm["Ws"][0]
        np_rest = [nW1[:128], nW1[128:], npm["bs"][0].reshape(1, -1),
                   npm["Ws"][1], npm["bs"][1].reshape(1, -1),
                   npm["Ws"][2], npm["bs"][2].reshape(1, -1),
                   npm["ln_scale"].reshape(1, -1),
                   npm["ln_bias"].reshape(1, -1)]
        x = _row_call(_make_node_body(2 * NH), n_nodes, N_NODE_BLK,
                      [x] + [part[h][i] for h in range(NH)
                             for i in range(2)],
                      np_rest, 128)

    out = _row_call(_dec_body, n_nodes, N_NODE_BLK, [x],
                    _mlp_params(params["node_dec"]), 3)
    return out


# speedup vs baseline: 2.3763x; 1.0003x over previous
import functools

import jax
import jax.numpy as jnp
from jax import lax
from jax.experimental import pallas as pl
from jax.experimental.pallas import tpu as pltpu
from jax.experimental.pallas import tpu_sc as plsc

INTERPRET = False

NW = 32
K = 128
NCHUNK = 80
C = NCHUNK * K
NE_PAD = NW * C

N_EDGE_BLK = 2560
N_NODE_BLK = 2000


def _ln(x, scale, bias):
    mu = jnp.mean(x, axis=-1, keepdims=True)
    var = jnp.mean((x - mu) ** 2, axis=-1, keepdims=True)
    return (x - mu) * jax.lax.rsqrt(var + 1e-5) * scale + bias


def _enc_body(x_ref, w1, b1, w2, b2, w3, b3, lns, lnb, o_ref):
    h = jnp.maximum(jnp.dot(x_ref[...], w1[...],
                            preferred_element_type=jnp.float32,
                           ) + b1[...], 0.0)
    h = jnp.maximum(jnp.dot(h, w2[...], preferred_element_type=jnp.float32,
                           ) + b2[...], 0.0)
    h = jnp.dot(h, w3[...], preferred_element_type=jnp.float32,
               ) + b3[...]
    o_ref[...] = _ln(h, lns[...], lnb[...])


def _dec_body(x_ref, w1, b1, w2, b2, w3, b3, o_ref):
    h = jnp.maximum(jnp.dot(x_ref[...], w1[...],
                            preferred_element_type=jnp.float32,
                           ) + b1[...], 0.0)
    h = jnp.maximum(jnp.dot(h, w2[...], preferred_element_type=jnp.float32,
                           ) + b2[...], 0.0)
    o_ref[...] = jnp.dot(h, w3[...], preferred_element_type=jnp.float32,
                        ) + b3[...]


def _edge_proc_body(e_ref, gs_ref, gd_ref, w1, w1s, w1d, b1, w2, b2, w3, b3,
                    lns, lnb, o_ref):
    h = jnp.maximum(jnp.dot(e_ref[...], w1[...],
                            preferred_element_type=jnp.float32)
                    + jnp.dot(gs_ref[...], w1s[...],
                              preferred_element_type=jnp.float32)
                    + jnp.dot(gd_ref[...], w1d[...],
                              preferred_element_type=jnp.float32)
                    + b1[...], 0.0)
    h = jnp.maximum(jnp.dot(h, w2[...], preferred_element_type=jnp.float32,
                           ) + b2[...], 0.0)
    h = jnp.dot(h, w3[...], preferred_element_type=jnp.float32,
               ) + b3[...]
    o_ref[...] = _ln(h, lns[...], lnb[...]) + e_ref[...]


def _make_node_body(nparts):
    def body(x_ref, *args):
        parts = args[:nparts]
        w1x, w1a, b1, w2, b2, w3, b3, lns, lnb, o_ref = args[nparts:]
        agg = parts[0][...]
        for p in parts[1:]:
            agg = agg + p[...]
        h = jnp.maximum(jnp.dot(x_ref[...], w1x[...],
                                preferred_element_type=jnp.float32)
                        + jnp.dot(agg, w1a[...],
                                  preferred_element_type=jnp.float32)
                        + b1[...], 0.0)
        h = jnp.maximum(jnp.dot(h, w2[...],
                                preferred_element_type=jnp.float32)
                        + b2[...], 0.0)
        h = jnp.dot(h, w3[...], preferred_element_type=jnp.float32) + b3[...]
        o_ref[...] = _ln(h, lns[...], lnb[...]) + x_ref[...]
    return body


def _row_call(body, n_rows, blk, row_args, full_args, out_dim):
    grid = (n_rows // blk,)
    in_specs = []
    for a in row_args:
        in_specs.append(pl.BlockSpec((blk, a.shape[1]), lambda i: (i, 0)))
    for a in full_args:
        in_specs.append(
            pl.BlockSpec(a.shape, lambda i: tuple(0 for _ in a.shape)))
    out_shape = jax.ShapeDtypeStruct((n_rows, out_dim), jnp.float32)
    out_spec = pl.BlockSpec((blk, out_dim), lambda i: (i, 0))
    return pl.pallas_call(
        body, grid=grid, in_specs=in_specs, out_specs=out_spec,
        out_shape=out_shape, interpret=INTERPRET,
    )(*row_args, *full_args)


def _make_sc_gather(nchunk, c):
    mesh = plsc.VectorSubcoreMesh(core_axis_name="c", subcore_axis_name="s")
    gshape = jax.ShapeDtypeStruct((NW * c, 128), jnp.float32)

    GROUPS = nchunk // 2

    @functools.partial(
        pl.kernel,
        out_type=[gshape, gshape],
        mesh=mesh,
        scratch_types=[
            pltpu.VMEM((nchunk, K), jnp.int32),
            pltpu.VMEM((nchunk, K), jnp.int32),
            pltpu.VMEM((K, 128), jnp.float32),
            pltpu.VMEM((K, 128), jnp.float32),
            pltpu.VMEM((K, 128), jnp.float32),
            pltpu.VMEM((K, 128), jnp.float32),
            pltpu.SemaphoreType.DMA,
            pltpu.SemaphoreType.DMA,
            pltpu.SemaphoreType.DMA,
            pltpu.SemaphoreType.DMA,
            pltpu.SemaphoreType.DMA,
            pltpu.SemaphoreType.DMA,
            pltpu.SemaphoreType.DMA,
            pltpu.SemaphoreType.DMA,
        ])
    def gather_k(xs_hbm, xd_hbm, src_hbm, dst_hbm, gs_hbm, gd_hbm,
                 src_v, dst_v, a0, b0, a1, b1,
                 sa0, sb0, sa1, sb1, oa0, ob0, oa1, ob1):
        cid = lax.axis_index("c")
        sid = lax.axis_index("s")
        wid = sid * 2 + cid
        pltpu.sync_copy(src_hbm.at[wid], src_v)
        pltpu.sync_copy(dst_hbm.at[wid], dst_v)
        base = wid * c

        def fire_g(j, ba, bb, sa, sb):
            pltpu.async_copy(xs_hbm.at[src_v.at[j]], ba, sa)
            pltpu.async_copy(xd_hbm.at[dst_v.at[j]], bb, sb)

        def wait_g(j, ba, bb, sa, sb):
            pltpu.make_async_copy(xs_hbm.at[src_v.at[j]], ba, sa).wait()
            pltpu.make_async_copy(xd_hbm.at[dst_v.at[j]], bb, sb).wait()

        def fire_o(j, ba, bb, oa, ob):
            rows = pl.ds(base + j * K, K)
            pltpu.async_copy(ba, gs_hbm.at[rows], oa)
            pltpu.async_copy(bb, gd_hbm.at[rows], ob)

        def wait_o(j, ba, bb, oa, ob):
            rows = pl.ds(base + j * K, K)
            pltpu.make_async_copy(ba, gs_hbm.at[rows], oa).wait()
            pltpu.make_async_copy(bb, gd_hbm.at[rows], ob).wait()

        fire_g(0, a0, b0, sa0, sb0)

        def gbody(g, _):
            j0 = 2 * g
            j1 = j0 + 1
            fire_g(j1, a1, b1, sa1, sb1)
            wait_g(j0, a0, b0, sa0, sb0)
            fire_o(j0, a0, b0, oa0, ob0)
            wait_g(j1, a1, b1, sa1, sb1)
            fire_o(j1, a1, b1, oa1, ob1)
            wait_o(j0, a0, b0, oa0, ob0)

            @pl.when(g < GROUPS - 1)
            def _():
                fire_g(j0 + 2, a0, b0, sa0, sb0)

            wait_o(j1, a1, b1, oa1, ob1)
            return 0

        lax.fori_loop(0, GROUPS, gbody, 0)

    return gather_k


def _make_sc_scatter(n_nodes, n_acc, nchunk, c):
    mesh = plsc.VectorSubcoreMesh(core_axis_name="c", subcore_axis_name="s")
    rows_per_tile = n_acc // 16
    nz_full, nz_rem = divmod(rows_per_tile, K)

    @functools.partial(
        pl.kernel,
        out_type=jax.ShapeDtypeStruct((2, n_acc, 128), jnp.float32),
        mesh=mesh,
        scratch_types=[
            pltpu.VMEM((nchunk, K), jnp.int32),
            pltpu.VMEM((K, 128), jnp.float32),
            pltpu.VMEM((K, 128), jnp.float32),
            pltpu.VMEM_SHARED((n_acc, 128), jnp.float32),
            pltpu.SemaphoreType.DMA,
            pltpu.SemaphoreType.DMA,
        ])
    def scatter_k(e_hbm, dst_hbm, out_hbm, dst_v, e0, e1, acc, s0, s1):
        cid = lax.axis_index("c")
        sid = lax.axis_index("s")
        wid = sid * 2 + cid
        pltpu.sync_copy(dst_hbm.at[wid], dst_v)

        def zrow(r, _):
            for c2 in range(8):
                e0[r, pl.ds(c2 * 16, 16)] = jnp.zeros((16,), jnp.float32)
            return 0

        lax.fori_loop(0, K, zrow, 0)
        for t in range(nz_full):
            pltpu.sync_copy(
                e0, acc.at[pl.ds(sid * rows_per_tile + t * K, K)])
        if nz_rem:
            pltpu.sync_copy(
                e0.at[pl.ds(0, nz_rem)],
                acc.at[pl.ds(sid * rows_per_tile + nz_full * K, nz_rem)])
        plsc.subcore_barrier()

        def fire_e(j, eb, sem):
            pltpu.async_copy(
                e_hbm.at[pl.ds(wid * c + j * K, K)], eb, sem)

        def wait_e(j, eb, sem):
            pltpu.make_async_copy(
                e_hbm.at[pl.ds(wid * c + j * K, K)], eb, sem).wait()

        fire_e(0, e0, s0)

        def sbody(g, _):
            j0 = 2 * g
            j1 = j0 + 1
            fire_e(j1, e1, s1)
            wait_e(j0, e0, s0)
            pltpu.sync_copy(e0, acc.at[dst_v.at[j0]], add=True)

            @pl.when(g < nchunk // 2 - 1)
            def _():
                fire_e(j0 + 2, e0, s0)

            wait_e(j1, e1, s1)
            pltpu.sync_copy(e1, acc.at[dst_v.at[j1]], add=True)
            return 0

        lax.fori_loop(0, nchunk // 2, sbody, 0)
        plsc.subcore_barrier()
        rows = pl.ds(sid * rows_per_tile, rows_per_tile)
        pltpu.sync_copy(acc.at[rows], out_hbm.at[cid, rows])

    return scatter_k


def _mlp_params(p):
    out = []
    for W, b in zip(p["Ws"], p["bs"]):
        out.append(W)
        out.append(b.reshape(1, -1))
    if "ln_scale" in p:
        out.append(p["ln_scale"].reshape(1, -1))
        out.append(p["ln_bias"].reshape(1, -1))
    return out


def kernel(node_features, edge_features, edge_index, params):
    n_nodes = node_features.shape[0]
    n_edges = edge_features.shape[0]
    n_acc = (n_nodes + 128) // 128 * 128
    src = edge_index[0]
    dst = edge_index[1]

    NH = 4
    NE_H = NE_PAD // NH
    NCHUNK_H = NCHUNK // NH
    C_H = NCHUNK_H * K
    pad = NE_PAD - n_edges
    zpad = jnp.zeros((pad,), jnp.int32)
    src_p = jnp.concatenate([src, zpad])
    dst_p = jnp.concatenate([dst, zpad])
    dst_sp = jnp.concatenate([dst, jnp.full((pad,), n_nodes, jnp.int32)])
    src_g = [src_p[h * NE_H:(h + 1) * NE_H].reshape(NW, NCHUNK_H, K)
             for h in range(NH)]
    dst_g = [dst_p[h * NE_H:(h + 1) * NE_H].reshape(NW, NCHUNK_H, K)
             for h in range(NH)]
    dst_s = [dst_sp[h * NE_H:(h + 1) * NE_H].reshape(NW, NCHUNK_H, K)
             for h in range(NH)]
    ef_pad = jnp.concatenate(
        [edge_features, jnp.zeros((pad, edge_features.shape[1]),
                                  jnp.float32)], axis=0)

    sc_gather = _make_sc_gather(NCHUNK_H, C_H)
    sc_scatter = _make_sc_scatter(n_nodes, n_acc, NCHUNK_H, C_H)

    e = [_row_call(_enc_body, NE_H, N_EDGE_BLK,
                   [ef_pad[h * NE_H:(h + 1) * NE_H]],
                   _mlp_params(params["edge_enc"]), 128)
         for h in range(NH)]
    x = _row_call(_enc_body, n_nodes, N_NODE_BLK, [node_features],
                  _mlp_params(params["node_enc"]), 128)

    for layer in params["processor"]:
        ep = layer["edge_mlp"]
        npm = layer["node_mlp"]
        W1 = ep["Ws"][0]
        ep_rest = [W1[:128], W1[128:256], W1[256:384],
                   ep["bs"][0].reshape(1, -1),
                   ep["Ws"][1], ep["bs"][1].reshape(1, -1),
                   ep["Ws"][2], ep["bs"][2].reshape(1, -1),
                   ep["ln_scale"].reshape(1, -1),
                   ep["ln_bias"].reshape(1, -1)]
        g = [sc_gather(x, x, src_g[h], dst_g[h]) for h in range(NH)]
        e = [_row_call(_edge_proc_body, NE_H, N_EDGE_BLK,
                       [e[h], g[h][0], g[h][1]], ep_rest, 128)
             for h in range(NH)]
        part = [sc_scatter(e[h], dst_s[h]) for h in range(NH)]
        nW1 = npm["Ws"][0]
        np_rest = [nW1[:128], nW1[128:], npm["bs"][0].reshape(1, -1),
                   npm["Ws"][1], npm["bs"][1].reshape(1, -1),
                   npm["Ws"][2], npm["bs"][2].reshape(1, -1),
                   npm["ln_scale"].reshape(1, -1),
                   npm["ln_bias"].reshape(1, -1)]
        x = _row_call(_make_node_body(2 * NH), n_nodes, N_NODE_BLK,
                      [x] + [part[h][i] for h in range(NH)
                             for i in range(2)],
                      np_rest, 128)

    out = _row_call(_dec_body, n_nodes, N_NODE_BLK, [x],
                    _mlp_params(params["node_dec"]), 3)
    return out
